# baseline (speedup 1.0000x reference)
import functools

import jax
import jax.numpy as jnp
from jax import lax
from jax.experimental import pallas as pl
from jax.experimental.pallas import tpu as pltpu
from jax.experimental.pallas import tpu_sc as plsc

D = 64
SCALE = 8.0
CB = 256


@functools.lru_cache(maxsize=None)
def _make_kernel(B, L, V):
    NC, NS = 2, 16
    NW = NC * NS
    n_pr = B // CB
    n_units = L * n_pr
    assert n_units % (2 * NW) == 0
    u_w = n_units // NW

    mesh = plsc.VectorSubcoreMesh(
        core_axis_name="c", subcore_axis_name="s", num_cores=NC, num_subcores=NS)

    buf_set = [
        pltpu.VMEM((1, CB), jnp.int32),
        pltpu.VMEM((1, CB), jnp.int32),
        pltpu.VMEM((CB, D), jnp.float32),
        pltpu.VMEM((CB, D), jnp.float32),
        pltpu.SemaphoreType.DMA,
        pltpu.SemaphoreType.DMA,
        pltpu.SemaphoreType.DMA,
    ]

    @functools.partial(
        pl.kernel,
        out_type=jax.ShapeDtypeStruct((B, L * D), jnp.float32),
        mesh=mesh,
        scratch_types=buf_set + buf_set + [pltpu.VMEM((L, D), jnp.float32)],
        compiler_params=pltpu.CompilerParams(
            use_tc_tiling_on_sc=False, needs_layout_passes=False,
            skip_device_barrier=True),
    )
    def body(w_hbm, p_hbm, emb0_hbm, emb1_hbm, out_hbm,
             wiA, piA, wrowA, obufA, semIA, semGA, semOA,
             wiB, piB, wrowB, obufB, semIB, semGB, semOB, emb1_v):
        wid = lax.axis_index("s") * NC + lax.axis_index("c")
        u0 = wid * u_w
        pltpu.sync_copy(emb1_hbm, emb1_v)

        def coords(u):
            l = u // n_pr
            b0 = pl.multiple_of((u % n_pr) * CB, CB)
            return l, b0

        def stage_idx(u, wi, pi, sem):
            l, b0 = coords(u)
            pltpu.async_copy(w_hbm.at[pl.ds(l, 1), pl.ds(b0, CB)], wi, sem)
            pltpu.async_copy(p_hbm.at[pl.ds(l, 1), pl.ds(b0, CB)], pi, sem)

        def wait_idx(wi, pi, sem):
            pltpu.make_async_copy(w_hbm.at[pl.ds(0, 1), pl.ds(0, CB)], wi, sem).wait()
            pltpu.make_async_copy(p_hbm.at[pl.ds(0, 1), pl.ds(0, CB)], pi, sem).wait()

        def fire_gathers(wi, pi, wrow, sem):
            for g in range(CB // 128):
                sl = pl.ds(g * 128, 128)
                pltpu.async_copy(emb0_hbm.at[wi.at[0, sl]], wrow.at[sl], sem)

        def wait_gathers(wi, pi, wrow, sem):
            for g in range(CB // 128):
                sl = pl.ds(g * 128, 128)
                pltpu.make_async_copy(emb0_hbm.at[wi.at[0, sl]], wrow.at[sl], sem).wait()

        def compute(wrow, pi, obuf):
            @plsc.parallel_loop(0, CB // 16)
            def r_body(r16):
                r0 = pl.multiple_of(r16 * 16, 16)
                pvec = pi[0, pl.ds(r0, 16)]
                for i in range(16):
                    p = pvec[i]
                    for c in range(D // 16):
                        sl = pl.ds(c * 16, 16)
                        obuf[r0 + i, sl] = wrow[r0 + i, sl] * SCALE + emb1_v[p, sl]

        def fire_out(u, obuf, sem):
            l, b0 = coords(u)
            loff = pl.multiple_of(l * D, D)
            pltpu.async_copy(obuf, out_hbm.at[pl.ds(b0, CB), pl.ds(loff, D)], sem)

        def wait_out(obuf, sem):
            pltpu.make_async_copy(
                obuf, out_hbm.at[pl.ds(0, CB), pl.ds(0, D)], sem).wait()

        stage_idx(u0, wiA, piA, semIA)
        wait_idx(wiA, piA, semIA)
        fire_gathers(wiA, piA, wrowA, semGA)
        stage_idx(u0 + 1, wiB, piB, semIB)

        def pair_body(j, carry):
            uA = u0 + 2 * j
            uB = uA + 1
            wait_idx(wiB, piB, semIB)
            fire_gathers(wiB, piB, wrowB, semGB)
            wait_gathers(wiA, piA, wrowA, semGA)

            @pl.when(j > 0)
            def _():
                wait_out(obufA, semOA)

            compute(wrowA, piA, obufA)
            fire_out(uA, obufA, semOA)

            @pl.when(j < u_w // 2 - 1)
            def _():
                stage_idx(uA + 2, wiA, piA, semIA)

            wait_gathers(wiB, piB, wrowB, semGB)

            @pl.when(j > 0)
            def _():
                wait_out(obufB, semOB)

            compute(wrowB, piB, obufB)
            fire_out(uB, obufB, semOB)

            @pl.when(j < u_w // 2 - 1)
            def _():
                wait_idx(wiA, piA, semIA)
                fire_gathers(wiA, piA, wrowA, semGA)
                stage_idx(uB + 2, wiB, piB, semIB)

            return carry

        lax.fori_loop(0, u_w // 2, pair_body, 0)
        wait_out(obufA, semOA)
        wait_out(obufB, semOB)

    return body


def kernel(src_word, src_pos, emb0_table, emb1_table):
    B, L, _ = src_word.shape
    V = emb0_table.shape[0]
    w2 = jnp.transpose(src_word.astype(jnp.int32), (1, 2, 0)).reshape(L, B)
    p2 = jnp.transpose(src_pos.astype(jnp.int32), (1, 2, 0)).reshape(L, B)
    out = _make_kernel(B, L, V)(w2, p2, emb0_table, emb1_table)
    return out.reshape(B, L, D)


# speedup vs baseline: 1.0026x; 1.0026x over previous
import functools

import jax
import jax.numpy as jnp
from jax import lax
from jax.experimental import pallas as pl
from jax.experimental.pallas import tpu as pltpu
from jax.experimental.pallas import tpu_sc as plsc

D = 64
SCALE = 8.0
CB = 256


@functools.lru_cache(maxsize=None)
def _make_kernel(B, L, V):
    NC, NS = 2, 16
    NW = NC * NS
    n_pr = B // CB
    n_units = L * n_pr
    assert n_units % (2 * NW) == 0
    u_w = n_units // NW

    mesh = plsc.VectorSubcoreMesh(
        core_axis_name="c", subcore_axis_name="s", num_cores=NC, num_subcores=NS)

    buf_set = [
        pltpu.VMEM((1, CB), jnp.int32),
        pltpu.VMEM((1, CB), jnp.int32),
        pltpu.VMEM((CB, D), jnp.float32),
        pltpu.VMEM((CB, D), jnp.float32),
        pltpu.SemaphoreType.DMA,
        pltpu.SemaphoreType.DMA,
        pltpu.SemaphoreType.DMA,
    ]

    @functools.partial(
        pl.kernel,
        out_type=jax.ShapeDtypeStruct((B, L * D), jnp.float32),
        mesh=mesh,
        scratch_types=buf_set + buf_set + [pltpu.VMEM((L, D), jnp.float32)],
        compiler_params=pltpu.CompilerParams(
            use_tc_tiling_on_sc=False, needs_layout_passes=False),
    )
    def body(w_hbm, p_hbm, emb0_hbm, emb1_hbm, out_hbm,
             wiA, piA, wrowA, obufA, semIA, semGA, semOA,
             wiB, piB, wrowB, obufB, semIB, semGB, semOB, emb1_v):
        wid = lax.axis_index("s") * NC + lax.axis_index("c")
        u0 = wid * u_w
        pltpu.sync_copy(emb1_hbm, emb1_v)

        def coords(u):
            l = u // n_pr
            b0 = pl.multiple_of((u % n_pr) * CB, CB)
            return l, b0

        def stage_idx(u, wi, pi, sem):
            l, b0 = coords(u)
            pltpu.async_copy(w_hbm.at[pl.ds(l, 1), pl.ds(b0, CB)], wi, sem)
            pltpu.async_copy(p_hbm.at[pl.ds(l, 1), pl.ds(b0, CB)], pi, sem)

        def wait_idx(wi, pi, sem):
            pltpu.make_async_copy(w_hbm.at[pl.ds(0, 1), pl.ds(0, CB)], wi, sem).wait()
            pltpu.make_async_copy(p_hbm.at[pl.ds(0, 1), pl.ds(0, CB)], pi, sem).wait()

        def fire_gathers(wi, pi, wrow, sem):
            for g in range(CB // 128):
                sl = pl.ds(g * 128, 128)
                pltpu.async_copy(emb0_hbm.at[wi.at[0, sl]], wrow.at[sl], sem)

        def wait_gathers(wi, pi, wrow, sem):
            for g in range(CB // 128):
                sl = pl.ds(g * 128, 128)
                pltpu.make_async_copy(emb0_hbm.at[wi.at[0, sl]], wrow.at[sl], sem).wait()

        def compute(wrow, pi, obuf):
            @plsc.parallel_loop(0, CB // 16)
            def r_body(r16):
                r0 = pl.multiple_of(r16 * 16, 16)
                pvec = pi[0, pl.ds(r0, 16)]
                for i in range(16):
                    p = pvec[i]
                    for c in range(D // 16):
                        sl = pl.ds(c * 16, 16)
                        obuf[r0 + i, sl] = wrow[r0 + i, sl] * SCALE + emb1_v[p, sl]

        def fire_out(u, obuf, sem):
            l, b0 = coords(u)
            loff = pl.multiple_of(l * D, D)
            pltpu.async_copy(obuf, out_hbm.at[pl.ds(b0, CB), pl.ds(loff, D)], sem)

        def wait_out(obuf, sem):
            pltpu.make_async_copy(
                obuf, out_hbm.at[pl.ds(0, CB), pl.ds(0, D)], sem).wait()

        stage_idx(u0, wiA, piA, semIA)
        wait_idx(wiA, piA, semIA)
        fire_gathers(wiA, piA, wrowA, semGA)
        stage_idx(u0 + 1, wiB, piB, semIB)

        def pair_body(j, carry):
            uA = u0 + 2 * j
            uB = uA + 1
            wait_idx(wiB, piB, semIB)
            fire_gathers(wiB, piB, wrowB, semGB)
            wait_gathers(wiA, piA, wrowA, semGA)

            @pl.when(j > 0)
            def _():
                wait_out(obufA, semOA)

            compute(wrowA, piA, obufA)
            fire_out(uA, obufA, semOA)

            @pl.when(j < u_w // 2 - 1)
            def _():
                stage_idx(uA + 2, wiA, piA, semIA)

            wait_gathers(wiB, piB, wrowB, semGB)

            @pl.when(j > 0)
            def _():
                wait_out(obufB, semOB)

            compute(wrowB, piB, obufB)
            fire_out(uB, obufB, semOB)

            @pl.when(j < u_w // 2 - 1)
            def _():
                wait_idx(wiA, piA, semIA)
                fire_gathers(wiA, piA, wrowA, semGA)
                stage_idx(uB + 2, wiB, piB, semIB)

            return carry

        lax.fori_loop(0, u_w // 2, pair_body, 0)
        wait_out(obufA, semOA)
        wait_out(obufB, semOB)

    return body


def kernel(src_word, src_pos, emb0_table, emb1_table):
    B, L, _ = src_word.shape
    V = emb0_table.shape[0]
    w2 = jnp.transpose(src_word.astype(jnp.int32), (1, 2, 0)).reshape(L, B)
    p2 = jnp.transpose(src_pos.astype(jnp.int32), (1, 2, 0)).reshape(L, B)
    out = _make_kernel(B, L, V)(w2, p2, emb0_table, emb1_table)
    return out.reshape(B, L, D)
